# baseline (speedup 1.0000x reference)
import functools

import jax
import jax.numpy as jnp
from jax import lax
from jax.experimental import pallas as pl
from jax.experimental.pallas import tpu as pltpu
from jax.experimental.pallas import tpu_sc as plsc

V_DIM = 100000
EMB = 128
BATCH = 16384
HIST = 50

NC, NS = 2, 16
NW = NC * NS
ROWS_PER_W = BATCH // NW
CHUNK = 8
NCHUNK = ROWS_PER_W // CHUNK
NBUF = 3
IDX_ROW = 80
GPC = CHUNK * HIST // IDX_ROW
LANES = 16
COLS = EMB // LANES
PAIRS = EMB // (2 * LANES)
PACKED = EMB // 2
SCALE = 1.0 / HIST

PACK_ROWS = 2000


def _pack_tc(x_ref, o_ref):
    x = x_ref[...]
    b = (x + ((x >> 16) & 1) + 0x7FFF) >> 16
    lo = b[:, :PACKED] & 0xFFFF
    hi = b[:, PACKED:] << 16
    o_ref[...] = lo | hi


_mesh = plsc.VectorSubcoreMesh(core_axis_name="c", subcore_axis_name="s")


@functools.partial(
    pl.kernel,
    out_type=jax.ShapeDtypeStruct((BATCH, EMB), jnp.float32),
    mesh=_mesh,
    scratch_types=[
        pltpu.VMEM((ROWS_PER_W * HIST // IDX_ROW, IDX_ROW), jnp.int32),
        pltpu.VMEM((NBUF, CHUNK * HIST, PACKED), jnp.int32),
        pltpu.VMEM((CHUNK, EMB), jnp.float32),
        pltpu.SemaphoreType.DMA,
        pltpu.SemaphoreType.DMA,
        pltpu.SemaphoreType.DMA,
    ],
    compiler_params=pltpu.CompilerParams(use_tc_tiling_on_sc=False),
)
def _cbow_sc(table_hbm, idx_hbm, out_hbm, idx_v, rows_v, outb,
             sem0, sem1, sem2):
    wid = lax.axis_index("s") * NC + lax.axis_index("c")
    sems = (sem0, sem1, sem2)
    idx_rows_per_w = ROWS_PER_W * HIST // IDX_ROW
    pltpu.sync_copy(idx_hbm.at[pl.ds(wid * idx_rows_per_w, idx_rows_per_w), :],
                    idx_v)

    mask = jnp.full((LANES,), jnp.int32(-65536))

    def gathers(i, b):
        return [
            pltpu.make_async_copy(
                table_hbm.at[idx_v.at[i * GPC + g]],
                rows_v.at[b, pl.ds(g * IDX_ROW, IDX_ROW), :],
                sems[b],
            )
            for g in range(GPC)
        ]

    def fire(i, b):
        for cp in gathers(i, b):
            cp.start()

    def drain(i, b):
        for cp in gathers(i, b):
            cp.wait()

    def compute(i, b):
        for r0 in range(CHUNK):
            def hbody(h, accs):
                r = r0 * HIST + h
                accs = list(accs)
                for c in range(PAIRS):
                    u = rows_v[b, r, pl.ds(c * LANES, LANES)]
                    lo = lax.bitcast_convert_type(u << 16, jnp.float32)
                    hi = lax.bitcast_convert_type(u & mask, jnp.float32)
                    accs[c] = accs[c] + lo
                    accs[PAIRS + c] = accs[PAIRS + c] + hi
                return tuple(accs)
            accs = lax.fori_loop(
                0, HIST, hbody,
                tuple(jnp.zeros((LANES,), jnp.float32) for _ in range(COLS)))
            for c in range(PAIRS):
                outb[r0, pl.ds(c * LANES, LANES)] = accs[c] * SCALE
                outb[r0, pl.ds(PACKED + c * LANES, LANES)] = (
                    accs[PAIRS + c] * SCALE)
        pltpu.sync_copy(outb,
                        out_hbm.at[pl.ds(wid * ROWS_PER_W + i * CHUNK, CHUNK), :])

    for p in range(NBUF):
        fire(p, p)

    @pl.loop(0, NCHUNK - (NCHUNK % NBUF), step=NBUF)
    def chunk(j):
        for b in range(NBUF):
            i = j + b
            drain(i, b)
            compute(i, b)
            nxt = i + NBUF
            @pl.when(nxt < NCHUNK)
            def _():
                fire(nxt, b)

    for t in range(NCHUNK % NBUF):
        i = NCHUNK - (NCHUNK % NBUF) + t
        drain(i, t)
        compute(i, t)


def kernel(inputs, table):
    idx = inputs.astype(jnp.int32).reshape(BATCH * HIST // IDX_ROW, IDX_ROW)
    t32 = lax.bitcast_convert_type(table, jnp.int32)
    packed = pl.pallas_call(
        _pack_tc,
        grid=(V_DIM // PACK_ROWS,),
        in_specs=[pl.BlockSpec((PACK_ROWS, EMB), lambda i: (i, 0))],
        out_specs=pl.BlockSpec((PACK_ROWS, PACKED), lambda i: (i, 0)),
        out_shape=jax.ShapeDtypeStruct((V_DIM, PACKED), jnp.int32),
    )(t32)
    return _cbow_sc(packed, idx)


# speedup vs baseline: 1.4911x; 1.4911x over previous
import functools

import jax
import jax.numpy as jnp
from jax import lax
from jax.experimental import pallas as pl
from jax.experimental.pallas import tpu as pltpu
from jax.experimental.pallas import tpu_sc as plsc

V_DIM = 100000
EMB = 128
BATCH = 16384
HIST = 50

NC, NS = 2, 16
NW = NC * NS
ROWS_PER_W = BATCH // NW
CHUNK = 4
NCHUNK = ROWS_PER_W // CHUNK
NBUF = 3
IDX_ROW = 100
GPC = CHUNK * HIST // IDX_ROW
LANES = 16
COLS = EMB // LANES
SCALE = 1.0 / HIST

_mesh = plsc.VectorSubcoreMesh(core_axis_name="c", subcore_axis_name="s")


@functools.partial(
    pl.kernel,
    out_type=jax.ShapeDtypeStruct((BATCH, EMB), jnp.float32),
    mesh=_mesh,
    scratch_types=[
        pltpu.VMEM((ROWS_PER_W * HIST // IDX_ROW, IDX_ROW), jnp.int32),
        pltpu.VMEM((NBUF, CHUNK * HIST, EMB), jnp.float32),
        pltpu.VMEM((NBUF, CHUNK, EMB), jnp.float32),
        pltpu.SemaphoreType.DMA,
        pltpu.SemaphoreType.DMA,
        pltpu.SemaphoreType.DMA,
        pltpu.SemaphoreType.DMA,
        pltpu.SemaphoreType.DMA,
        pltpu.SemaphoreType.DMA,
    ],
)
def _cbow_sc(table_hbm, idx_hbm, out_hbm, idx_v, rows_v, outb,
             sem0, sem1, sem2, osem0, osem1, osem2):
    wid = lax.axis_index("s") * NC + lax.axis_index("c")
    sems = (sem0, sem1, sem2)
    osems = (osem0, osem1, osem2)
    idx_rows_per_w = ROWS_PER_W * HIST // IDX_ROW
    pltpu.sync_copy(idx_hbm.at[pl.ds(wid * idx_rows_per_w, idx_rows_per_w), :],
                    idx_v)

    def gathers(i, b):
        return [
            pltpu.make_async_copy(
                table_hbm.at[idx_v.at[i * GPC + g]],
                rows_v.at[b, pl.ds(g * IDX_ROW, IDX_ROW), :],
                sems[b],
            )
            for g in range(GPC)
        ]

    def out_copy(i, b):
        return pltpu.make_async_copy(
            outb.at[b],
            out_hbm.at[pl.ds(wid * ROWS_PER_W + i * CHUNK, CHUNK), :],
            osems[b],
        )

    def fire(i, b):
        for cp in gathers(i, b):
            cp.start()

    def drain(i, b):
        for cp in gathers(i, b):
            cp.wait()

    def compute(i, b, first_round):
        if not first_round:
            out_copy(i - NBUF, b).wait()
        for r0 in range(CHUNK):
            def hbody(h, accs):
                r = r0 * HIST + h
                return tuple(accs[c] + rows_v[b, r, pl.ds(c * LANES, LANES)]
                             for c in range(COLS))
            accs = lax.fori_loop(
                0, HIST, hbody,
                tuple(jnp.zeros((LANES,), jnp.float32) for _ in range(COLS)))
            for c in range(COLS):
                outb[b, r0, pl.ds(c * LANES, LANES)] = accs[c] * SCALE
        out_copy(i, b).start()

    for p in range(NBUF):
        fire(p, p)

    for b in range(NBUF):
        drain(b, b)
        compute(b, b, True)
        fire(b + NBUF, b)

    @pl.loop(NBUF, NCHUNK - (NCHUNK % NBUF), step=NBUF)
    def chunk(j):
        for b in range(NBUF):
            i = j + b
            drain(i, b)
            compute(i, b, False)
            nxt = i + NBUF
            @pl.when(nxt < NCHUNK)
            def _():
                fire(nxt, b)

    for t in range(NCHUNK % NBUF):
        i = NCHUNK - (NCHUNK % NBUF) + t
        drain(i, t)
        compute(i, t, False)

    for t in range(NBUF):
        i = NCHUNK - NBUF + t
        b = i % NBUF
        out_copy(i, b).wait()


def kernel(inputs, table):
    idx = inputs.astype(jnp.int32).reshape(BATCH * HIST // IDX_ROW, IDX_ROW)
    return _cbow_sc(table, idx)
